# baseline (speedup 1.0000x reference)
import functools

import jax
import jax.numpy as jnp
from jax import lax
from jax.experimental import pallas as pl
from jax.experimental.pallas import tpu as pltpu
from jax.experimental.pallas import tpu_sc as plsc

_CHUNK = 4096


@functools.lru_cache(maxsize=None)
def _build(B, V, D):
    info = plsc.get_sparse_core_info()
    nw = info.num_cores * info.num_subcores
    rows_per_w = D // nw
    n_chunks = B // _CHUNK
    mesh = plsc.VectorSubcoreMesh(core_axis_name="c", subcore_axis_name="s")

    @functools.partial(
        pl.kernel,
        mesh=mesh,
        out_type=jax.ShapeDtypeStruct((D, B), jnp.float32),
        compiler_params=pltpu.CompilerParams(needs_layout_passes=False),
        scratch_types=[
            pltpu.VMEM((B,), jnp.int32),
            pltpu.VMEM((V,), jnp.float32),
            pltpu.VMEM((_CHUNK,), jnp.float32),
            pltpu.VMEM((_CHUNK,), jnp.float32),
            pltpu.SemaphoreType.DMA,
            pltpu.SemaphoreType.DMA,
            pltpu.SemaphoreType.DMA,
        ],
    )
    def gather_kernel(idx_hbm, table_hbm, out_hbm, ids_v, row_v, out_a, out_b,
                      sem_ids, sem_row, sem_out):
        def start_row_copy(d):
            return [pltpu.async_copy(table_hbm.at[d], row_v, sem_row)]

        out_bufs = (out_a, out_b)
        wid = lax.axis_index("s") * info.num_cores + lax.axis_index("c")
        ids_cp = pltpu.async_copy(idx_hbm, ids_v, sem_ids)
        row_cps = start_row_copy(wid * rows_per_w)
        ids_cp.wait()
        for rr in range(rows_per_w):
            d = wid * rows_per_w + rr
            for cp in row_cps:
                cp.wait()
            out_cps = [None, None]
            for c in range(n_chunks):
                ob = out_bufs[c % 2]
                if out_cps[c % 2] is not None:
                    out_cps[c % 2].wait()

                @plsc.parallel_loop(0, _CHUNK // 16, unroll=8)
                def body(i):
                    idxv = ids_v[pl.ds(c * _CHUNK + i * 16, 16)]
                    ob[pl.ds(i * 16, 16)] = plsc.load_gather(row_v, [idxv])

                out_cps[c % 2] = pltpu.async_copy(
                    ob, out_hbm.at[d, pl.ds(c * _CHUNK, _CHUNK)], sem_out)
            if rr + 1 < rows_per_w:
                row_cps = start_row_copy(d + 1)
            for cp in out_cps:
                cp.wait()

    return gather_kernel


def kernel(class_ids, table):
    (B,) = class_ids.shape
    V, D = table.shape
    gather_kernel = _build(B, V, D)
    out_t = gather_kernel(class_ids.astype(jnp.int32), table.T)
    return out_t.T


# speedup vs baseline: 1.0015x; 1.0015x over previous
import functools

import jax
import jax.numpy as jnp
from jax import lax
from jax.experimental import pallas as pl
from jax.experimental.pallas import tpu as pltpu
from jax.experimental.pallas import tpu_sc as plsc

_CHUNK = 4096


@functools.lru_cache(maxsize=None)
def _build(B, V, D):
    info = plsc.get_sparse_core_info()
    nw = info.num_cores * info.num_subcores
    rows_per_w = D // nw
    n_chunks = B // _CHUNK
    mesh = plsc.VectorSubcoreMesh(core_axis_name="c", subcore_axis_name="s")

    @functools.partial(
        pl.kernel,
        mesh=mesh,
        out_type=jax.ShapeDtypeStruct((D, B), jnp.float32),
        compiler_params=pltpu.CompilerParams(needs_layout_passes=False),
        scratch_types=[
            pltpu.VMEM((B,), jnp.int32),
            pltpu.VMEM((V,), jnp.float32),
            pltpu.VMEM((_CHUNK,), jnp.float32),
            pltpu.VMEM((_CHUNK,), jnp.float32),
            pltpu.SemaphoreType.DMA,
            pltpu.SemaphoreType.DMA,
            pltpu.SemaphoreType.DMA,
        ],
    )
    def gather_kernel(idx_hbm, table_hbm, out_hbm, ids_v, row_v, out_a, out_b,
                      sem_ids, sem_row, sem_out):
        out_bufs = (out_a, out_b)
        wid = lax.axis_index("s") * info.num_cores + lax.axis_index("c")
        ids_cp = pltpu.async_copy(idx_hbm, ids_v, sem_ids)
        row_cp = pltpu.async_copy(table_hbm.at[wid * rows_per_w], row_v, sem_row)
        ids_cp.wait()
        for rr in range(rows_per_w):
            d = wid * rows_per_w + rr
            row_cp.wait()
            out_cps = [None, None]
            for c in range(n_chunks):
                ob = out_bufs[c % 2]
                if out_cps[c % 2] is not None:
                    out_cps[c % 2].wait()

                @plsc.parallel_loop(0, _CHUNK // 16, unroll=8)
                def body(i):
                    idxv = ids_v[pl.ds(c * _CHUNK + i * 16, 16)]
                    ob[pl.ds(i * 16, 16)] = plsc.load_gather(row_v, [idxv])

                out_cps[c % 2] = pltpu.async_copy(
                    ob, out_hbm.at[d, pl.ds(c * _CHUNK, _CHUNK)], sem_out)
            if rr + 1 < rows_per_w:
                row_cp = pltpu.async_copy(table_hbm.at[d + 1], row_v, sem_row)
            for cp in out_cps:
                cp.wait()

    return gather_kernel


def kernel(class_ids, table):
    (B,) = class_ids.shape
    V, D = table.shape
    gather_kernel = _build(B, V, D)
    out_t = gather_kernel(class_ids.astype(jnp.int32), table.T)
    return out_t.T
